# Initial kernel scaffold; baseline (speedup 1.0000x reference)
import jax
import jax.numpy as jnp
from jax.experimental import pallas as pl


def kernel(x, edge_index, Wl1, bl1, Wr1, Wl2, bl2, Wr2):
    raise NotImplementedError("write your pallas kernel here")


# speedup vs baseline: 4.6793x; 4.6793x over previous
import functools

import jax
import jax.numpy as jnp
from jax import lax
from jax.experimental import pallas as pl
from jax.experimental.pallas import tpu as pltpu
from jax.experimental.pallas import tpu_sc as plsc

_N = 10000
_E = 320000
_D = 128

_NTILES = 32
_NSUB = 16
_N_PAD = 10240
_ROWS_PER_TILE = _N_PAD // _NSUB
_EDGES_PER_TILE = _E // _NTILES
_CHUNK = 80
_N_CHUNKS = _EDGES_PER_TILE // _CHUNK
_DEG_W = 16


def _make_sc_agg(with_deg: bool):
    mesh = plsc.VectorSubcoreMesh(core_axis_name="c", subcore_axis_name="s")
    out_type = [jax.ShapeDtypeStruct((2, _N_PAD, _D), jnp.float32)]
    scratch = [
        pltpu.VMEM((_CHUNK,), jnp.int32),
        pltpu.VMEM((_CHUNK,), jnp.int32),
        pltpu.VMEM((_CHUNK, _D), jnp.float32),
        pltpu.VMEM_SHARED((_N_PAD, _D), jnp.float32),
        pltpu.SemaphoreType.DMA,
    ]
    if with_deg:
        out_type.append(jax.ShapeDtypeStruct((2, _N_PAD, _D), jnp.float32))

    def body(y, src, dst, *rest):
        if with_deg:
            agg_out, deg_out, sidx, didx, rows, agg_sh, sem = rest
        else:
            agg_out, sidx, didx, rows, agg_sh, sem = rest
        cid = lax.axis_index("c")
        sid = lax.axis_index("s")
        tid = cid * _NSUB + sid
        ebase = tid * _EDGES_PER_TILE
        slab = sid * _ROWS_PER_TILE

        def fill_rows(val):
            def fill_row(r, carry):
                for c in range(_D // 16):
                    rows[r, pl.ds(c * 16, 16)] = jnp.full((16,), val,
                                                          jnp.float32)
                return carry

            lax.fori_loop(0, _CHUNK, fill_row, 0)

        def zero_slab():
            for j in range(_ROWS_PER_TILE // _CHUNK):
                pltpu.sync_copy(rows,
                                agg_sh.at[pl.ds(slab + j * _CHUNK, _CHUNK), :])

        def copy_slab_out(dst_hbm):
            pltpu.sync_copy(agg_sh.at[pl.ds(slab, _ROWS_PER_TILE), :],
                            dst_hbm.at[cid, pl.ds(slab, _ROWS_PER_TILE), :])

        fill_rows(0.0)
        zero_slab()
        plsc.subcore_barrier()

        def chunk(i, carry):
            base = pl.multiple_of(ebase + i * _CHUNK, 8)
            pltpu.sync_copy(src.at[pl.ds(base, _CHUNK)], sidx)
            pltpu.sync_copy(dst.at[pl.ds(base, _CHUNK)], didx)
            pltpu.async_copy(y.at[sidx], rows, sem).wait()
            pltpu.sync_copy(rows, agg_sh.at[didx], add=True)
            return carry

        lax.fori_loop(0, _N_CHUNKS, chunk, 0)
        plsc.subcore_barrier()
        copy_slab_out(agg_out)

        if with_deg:
            plsc.subcore_barrier()
            fill_rows(0.0)
            zero_slab()
            plsc.subcore_barrier()
            fill_rows(1.0)

            def deg_chunk(i, carry):
                base = pl.multiple_of(ebase + i * _CHUNK, 8)
                pltpu.sync_copy(dst.at[pl.ds(base, _CHUNK)], didx)
                pltpu.sync_copy(rows, agg_sh.at[didx], add=True)
                return carry

            lax.fori_loop(0, _N_CHUNKS, deg_chunk, 0)
            plsc.subcore_barrier()
            copy_slab_out(deg_out)

    return pl.kernel(body, mesh=mesh, out_type=out_type, scratch_types=scratch)


_sc_agg_deg = _make_sc_agg(with_deg=True)
_sc_agg = _make_sc_agg(with_deg=False)

_BLK = 1000
_GRID = _N // _BLK


def _lin2_body(x_ref, wl_ref, wr_ref, b_ref, y_ref, r_ref):
    xb = x_ref[...]
    y_ref[...] = jnp.dot(xb, wl_ref[...], preferred_element_type=jnp.float32)
    r_ref[...] = jnp.dot(xb, wr_ref[...],
                         preferred_element_type=jnp.float32) + b_ref[...]


def _mid_body(a0_ref, a1_ref, d0_ref, d1_ref, r1_ref, wl_ref, wr_ref, b_ref,
              y2_ref, r2_ref):
    deg = jnp.maximum(d0_ref[:, 0:1] + d1_ref[:, 0:1], 1.0)
    h = jnp.maximum((a0_ref[...] + a1_ref[...]) / deg + r1_ref[...], 0.0)
    y2_ref[...] = jnp.dot(h, wl_ref[...], preferred_element_type=jnp.float32)
    r2_ref[...] = jnp.dot(h, wr_ref[...],
                          preferred_element_type=jnp.float32) + b_ref[...]


def _out_body(a0_ref, a1_ref, d0_ref, d1_ref, r2_ref, z_ref):
    deg = jnp.maximum(d0_ref[:, 0:1] + d1_ref[:, 0:1], 1.0)
    z_ref[...] = (a0_ref[...] + a1_ref[...]) / deg + r2_ref[...]


def _row_spec(w=_D):
    return pl.BlockSpec((_BLK, w), lambda i: (i, 0))


def _full_spec(shape):
    return pl.BlockSpec(shape, lambda i: (0, 0))


_f32 = jnp.float32
_lin2 = pl.pallas_call(
    _lin2_body,
    grid=(_GRID,),
    in_specs=[_row_spec(), _full_spec((_D, _D)), _full_spec((_D, _D)),
              _full_spec((1, _D))],
    out_specs=[_row_spec(), _row_spec()],
    out_shape=[jax.ShapeDtypeStruct((_N, _D), _f32)] * 2,
)

_mid = pl.pallas_call(
    _mid_body,
    grid=(_GRID,),
    in_specs=[_row_spec(), _row_spec(), _row_spec(), _row_spec(),
              _row_spec(), _full_spec((_D, _D)), _full_spec((_D, _D)),
              _full_spec((1, _D))],
    out_specs=[_row_spec(), _row_spec()],
    out_shape=[jax.ShapeDtypeStruct((_N, _D), _f32)] * 2,
)

_out = pl.pallas_call(
    _out_body,
    grid=(_GRID,),
    in_specs=[_row_spec(), _row_spec(), _row_spec(), _row_spec(),
              _row_spec()],
    out_specs=_row_spec(),
    out_shape=jax.ShapeDtypeStruct((_N, _D), _f32),
)


def kernel(x, edge_index, Wl1, bl1, Wr1, Wl2, bl2, Wr2):
    src = edge_index[0]
    dst = edge_index[1]
    y1, r1 = _lin2(x, Wl1.T, Wr1.T, bl1.reshape(1, _D))
    agg1, deg2 = _sc_agg_deg(y1, src, dst)
    d0, d1 = deg2[0, :_N], deg2[1, :_N]
    y2, r2 = _mid(agg1[0, :_N], agg1[1, :_N], d0, d1, r1,
                  Wl2.T, Wr2.T, bl2.reshape(1, _D))
    agg2 = _sc_agg(y2, src, dst)
    if isinstance(agg2, (list, tuple)):
        agg2 = agg2[0]
    return _out(agg2[0, :_N], agg2[1, :_N], d0, d1, r2)


# speedup vs baseline: 9.8583x; 2.1068x over previous
import functools

import jax
import jax.numpy as jnp
from jax import lax
from jax.experimental import pallas as pl
from jax.experimental.pallas import tpu as pltpu
from jax.experimental.pallas import tpu_sc as plsc

_N = 10000
_E = 320000
_D = 128

_NTILES = 32
_NSUB = 16
_N_PAD = 10240
_ROWS_PER_TILE = _N_PAD // _NSUB
_EDGES_PER_TILE = _E // _NTILES
_CHUNK = 80
_N_CHUNKS = _EDGES_PER_TILE // _CHUNK
_DEG_W = 16


def _make_sc_agg(with_deg: bool):
    mesh = plsc.VectorSubcoreMesh(core_axis_name="c", subcore_axis_name="s")
    out_type = [jax.ShapeDtypeStruct((2, _N_PAD, _D), jnp.float32)]
    scratch = [
        pltpu.VMEM((_EDGES_PER_TILE,), jnp.int32),
        pltpu.VMEM((_CHUNK,), jnp.int32),
        pltpu.VMEM((_CHUNK,), jnp.int32),
        pltpu.VMEM((_CHUNK, _D), jnp.float32),
        pltpu.VMEM((_CHUNK, _D), jnp.float32),
        pltpu.VMEM_SHARED((_N_PAD, _D), jnp.float32),
        pltpu.SemaphoreType.DMA,
        pltpu.SemaphoreType.DMA,
        pltpu.SemaphoreType.DMA,
        pltpu.SemaphoreType.DMA,
    ]
    if with_deg:
        out_type.append(jax.ShapeDtypeStruct((2, _N_PAD, _D), jnp.float32))

    def body(y, src, dst, *rest):
        if with_deg:
            agg_out, deg_out = rest[:2]
            rest = rest[2:]
        else:
            agg_out = rest[0]
            rest = rest[1:]
        sbuf, didx0, didx1, rows0, rows1, agg_sh, sg0, sg1, si0, si1 = rest
        didx = (didx0, didx1)
        rows = (rows0, rows1)
        sg = (sg0, sg1)
        si = (si0, si1)
        cid = lax.axis_index("c")
        sid = lax.axis_index("s")
        tid = cid * _NSUB + sid
        ebase = tid * _EDGES_PER_TILE
        slab = sid * _ROWS_PER_TILE

        def fill_rows(val):
            def fill_row(r, carry):
                for c in range(_D // 16):
                    rows0[r, pl.ds(c * 16, 16)] = jnp.full((16,), val,
                                                           jnp.float32)
                return carry

            lax.fori_loop(0, _CHUNK, fill_row, 0)

        def zero_slab():
            for j in range(_ROWS_PER_TILE // _CHUNK):
                pltpu.sync_copy(rows0,
                                agg_sh.at[pl.ds(slab + j * _CHUNK, _CHUNK), :])

        def copy_slab_out(dst_hbm):
            pltpu.sync_copy(agg_sh.at[pl.ds(slab, _ROWS_PER_TILE), :],
                            dst_hbm.at[cid, pl.ds(slab, _ROWS_PER_TILE), :])

        def idx_start(c, b):
            base = pl.multiple_of(ebase + c * _CHUNK, 8)
            return pltpu.make_async_copy(dst.at[pl.ds(base, _CHUNK)],
                                         didx[b], si[b])

        def gather_start(c, b):
            sl = pl.multiple_of(c * _CHUNK, 8)
            return pltpu.make_async_copy(
                y.at[sbuf.at[pl.ds(sl, _CHUNK)]], rows[b], sg[b])

        def scatter(b):
            pltpu.sync_copy(rows[b], agg_sh.at[didx[b]], add=True)

        fill_rows(0.0)
        zero_slab()
        pltpu.sync_copy(src.at[pl.ds(pl.multiple_of(ebase, 8),
                                     _EDGES_PER_TILE)], sbuf)
        plsc.subcore_barrier()

        idx_start(0, 0).start()
        gather_start(0, 0).start()

        def pair(j, carry):
            c0 = j * 2
            idx_start(c0 + 1, 1).start()
            gather_start(c0 + 1, 1).start()
            gather_start(c0, 0).wait()
            idx_start(c0, 0).wait()
            scatter(0)
            idx_start(c0 + 2, 0).start()
            gather_start(c0 + 2, 0).start()
            gather_start(c0 + 1, 1).wait()
            idx_start(c0 + 1, 1).wait()
            scatter(1)
            return carry

        lax.fori_loop(0, (_N_CHUNKS - 1) // 2, pair, 0)
        gather_start(_N_CHUNKS - 1, 0).wait()
        idx_start(_N_CHUNKS - 1, 0).wait()
        scatter(0)
        plsc.subcore_barrier()
        copy_slab_out(agg_out)

        if with_deg:
            plsc.subcore_barrier()
            fill_rows(0.0)
            zero_slab()
            plsc.subcore_barrier()
            fill_rows(1.0)

            def deg_scatter(b):
                pltpu.sync_copy(rows0, agg_sh.at[didx[b]], add=True)

            idx_start(0, 0).start()

            def deg_pair(j, carry):
                c0 = j * 2
                idx_start(c0 + 1, 1).start()
                idx_start(c0, 0).wait()
                deg_scatter(0)
                idx_start(c0 + 2, 0).start()
                idx_start(c0 + 1, 1).wait()
                deg_scatter(1)
                return carry

            lax.fori_loop(0, (_N_CHUNKS - 1) // 2, deg_pair, 0)
            idx_start(_N_CHUNKS - 1, 0).wait()
            deg_scatter(0)
            plsc.subcore_barrier()
            copy_slab_out(deg_out)

    return pl.kernel(body, mesh=mesh, out_type=out_type, scratch_types=scratch)


_sc_agg_deg = _make_sc_agg(with_deg=True)
_sc_agg = _make_sc_agg(with_deg=False)

_BLK = 1000
_GRID = _N // _BLK


def _lin2_body(x_ref, wl_ref, wr_ref, b_ref, y_ref, r_ref):
    xb = x_ref[...]
    y_ref[...] = jnp.dot(xb, wl_ref[...], preferred_element_type=jnp.float32)
    r_ref[...] = jnp.dot(xb, wr_ref[...],
                         preferred_element_type=jnp.float32) + b_ref[...]


def _mid_body(a0_ref, a1_ref, d0_ref, d1_ref, r1_ref, wl_ref, wr_ref, b_ref,
              y2_ref, r2_ref):
    deg = jnp.maximum(d0_ref[:, 0:1] + d1_ref[:, 0:1], 1.0)
    h = jnp.maximum((a0_ref[...] + a1_ref[...]) / deg + r1_ref[...], 0.0)
    y2_ref[...] = jnp.dot(h, wl_ref[...], preferred_element_type=jnp.float32)
    r2_ref[...] = jnp.dot(h, wr_ref[...],
                          preferred_element_type=jnp.float32) + b_ref[...]


def _out_body(a0_ref, a1_ref, d0_ref, d1_ref, r2_ref, z_ref):
    deg = jnp.maximum(d0_ref[:, 0:1] + d1_ref[:, 0:1], 1.0)
    z_ref[...] = (a0_ref[...] + a1_ref[...]) / deg + r2_ref[...]


def _row_spec(w=_D):
    return pl.BlockSpec((_BLK, w), lambda i: (i, 0))


def _full_spec(shape):
    return pl.BlockSpec(shape, lambda i: (0, 0))


_f32 = jnp.float32
_lin2 = pl.pallas_call(
    _lin2_body,
    grid=(_GRID,),
    in_specs=[_row_spec(), _full_spec((_D, _D)), _full_spec((_D, _D)),
              _full_spec((1, _D))],
    out_specs=[_row_spec(), _row_spec()],
    out_shape=[jax.ShapeDtypeStruct((_N, _D), _f32)] * 2,
)

_mid = pl.pallas_call(
    _mid_body,
    grid=(_GRID,),
    in_specs=[_row_spec(), _row_spec(), _row_spec(), _row_spec(),
              _row_spec(), _full_spec((_D, _D)), _full_spec((_D, _D)),
              _full_spec((1, _D))],
    out_specs=[_row_spec(), _row_spec()],
    out_shape=[jax.ShapeDtypeStruct((_N, _D), _f32)] * 2,
)

_out = pl.pallas_call(
    _out_body,
    grid=(_GRID,),
    in_specs=[_row_spec(), _row_spec(), _row_spec(), _row_spec(),
              _row_spec()],
    out_specs=_row_spec(),
    out_shape=jax.ShapeDtypeStruct((_N, _D), _f32),
)


def kernel(x, edge_index, Wl1, bl1, Wr1, Wl2, bl2, Wr2):
    src = edge_index[0]
    dst = edge_index[1]
    y1, r1 = _lin2(x, Wl1.T, Wr1.T, bl1.reshape(1, _D))
    agg1, deg2 = _sc_agg_deg(y1, src, dst)
    d0, d1 = deg2[0, :_N], deg2[1, :_N]
    y2, r2 = _mid(agg1[0, :_N], agg1[1, :_N], d0, d1, r1,
                  Wl2.T, Wr2.T, bl2.reshape(1, _D))
    agg2 = _sc_agg(y2, src, dst)
    if isinstance(agg2, (list, tuple)):
        agg2 = agg2[0]
    return _out(agg2[0, :_N], agg2[1, :_N], d0, d1, r2)


# speedup vs baseline: 11.0656x; 1.1225x over previous
import functools

import jax
import jax.numpy as jnp
from jax import lax
from jax.experimental import pallas as pl
from jax.experimental.pallas import tpu as pltpu
from jax.experimental.pallas import tpu_sc as plsc

_N = 10000
_E = 320000
_D = 128

_NTILES = 32
_NSUB = 16
_N_PAD = 10240
_ROWS_PER_TILE = _N_PAD // _NSUB
_CHUNK = 128
_TOT_CHUNKS = _E // _CHUNK
_BASE_CHUNKS = _TOT_CHUNKS // _NTILES
_XTRA_TILES = _TOT_CHUNKS - _BASE_CHUNKS * _NTILES


def _make_sc_agg(with_deg: bool):
    mesh = plsc.VectorSubcoreMesh(core_axis_name="c", subcore_axis_name="s")
    out_type = [jax.ShapeDtypeStruct((2, _N_PAD, _D), jnp.float32)]
    scratch = [
        pltpu.VMEM(((_BASE_CHUNKS + 1) * _CHUNK,), jnp.int32),
        pltpu.VMEM((_CHUNK,), jnp.int32),
        pltpu.VMEM((_CHUNK,), jnp.int32),
        pltpu.VMEM((_CHUNK, _D), jnp.float32),
        pltpu.VMEM((_CHUNK, _D), jnp.float32),
        pltpu.VMEM_SHARED((_N_PAD, _D), jnp.float32),
        pltpu.SemaphoreType.DMA,
        pltpu.SemaphoreType.DMA,
        pltpu.SemaphoreType.DMA,
        pltpu.SemaphoreType.DMA,
    ]
    if with_deg:
        out_type.append(jax.ShapeDtypeStruct((2, _N_PAD, _D), jnp.float32))

    def body(y, src, dst, *rest):
        if with_deg:
            agg_out, deg_out = rest[:2]
            rest = rest[2:]
        else:
            agg_out = rest[0]
            rest = rest[1:]
        sbuf, didx0, didx1, rows0, rows1, agg_sh, sg0, sg1, si0, si1 = rest
        didx = (didx0, didx1)
        rows = (rows0, rows1)
        sg = (sg0, sg1)
        si = (si0, si1)
        cid = lax.axis_index("c")
        sid = lax.axis_index("s")
        tid = cid * _NSUB + sid
        chunk_base = tid * _BASE_CHUNKS + jnp.minimum(tid, _XTRA_TILES)
        n_c = _BASE_CHUNKS + (tid < _XTRA_TILES).astype(jnp.int32)
        has_extra = tid < _XTRA_TILES
        ebase = chunk_base * _CHUNK
        slab = sid * _ROWS_PER_TILE

        def fill_rows(val):
            def fill_row(r, carry):
                for c in range(_D // 16):
                    rows0[r, pl.ds(c * 16, 16)] = jnp.full((16,), val,
                                                           jnp.float32)
                return carry

            lax.fori_loop(0, _CHUNK, fill_row, 0)

        def zero_slab():
            for j in range(_ROWS_PER_TILE // _CHUNK):
                pltpu.sync_copy(rows0,
                                agg_sh.at[pl.ds(slab + j * _CHUNK, _CHUNK), :])

        def copy_slab_out(dst_hbm):
            pltpu.sync_copy(agg_sh.at[pl.ds(slab, _ROWS_PER_TILE), :],
                            dst_hbm.at[cid, pl.ds(slab, _ROWS_PER_TILE), :])

        def idx_start(c, b):
            base = pl.multiple_of(ebase + c * _CHUNK, 8)
            return pltpu.make_async_copy(dst.at[pl.ds(base, _CHUNK)],
                                         didx[b], si[b])

        def gather_start(c, b):
            sl = pl.multiple_of(c * _CHUNK, 8)
            return pltpu.make_async_copy(
                y.at[sbuf.at[pl.ds(sl, _CHUNK)]], rows[b], sg[b])

        def scatter(b):
            pltpu.sync_copy(rows[b], agg_sh.at[didx[b]], add=True)

        fill_rows(0.0)
        zero_slab()
        nbulk = _BASE_CHUNKS * _CHUNK
        pltpu.sync_copy(src.at[pl.ds(pl.multiple_of(ebase, 8), nbulk)],
                        sbuf.at[pl.ds(0, nbulk)])

        @pl.when(has_extra)
        def _():
            pltpu.sync_copy(
                src.at[pl.ds(pl.multiple_of(ebase + nbulk, 8), _CHUNK)],
                sbuf.at[pl.ds(nbulk, _CHUNK)])

        plsc.subcore_barrier()

        idx_start(0, 0).start()
        gather_start(0, 0).start()

        def pair(j, carry):
            c0 = j * 2
            idx_start(c0 + 1, 1).start()
            gather_start(c0 + 1, 1).start()
            gather_start(c0, 0).wait()
            idx_start(c0, 0).wait()
            scatter(0)

            @pl.when(c0 + 2 < n_c)
            def _():
                idx_start(c0 + 2, 0).start()
                gather_start(c0 + 2, 0).start()

            gather_start(c0 + 1, 1).wait()
            idx_start(c0 + 1, 1).wait()
            scatter(1)
            return carry

        lax.fori_loop(0, _BASE_CHUNKS // 2, pair, 0)

        @pl.when(has_extra)
        def _():
            gather_start(_BASE_CHUNKS, 0).wait()
            idx_start(_BASE_CHUNKS, 0).wait()
            scatter(0)

        plsc.subcore_barrier()
        copy_slab_out(agg_out)

        if with_deg:
            plsc.subcore_barrier()
            fill_rows(0.0)
            zero_slab()
            plsc.subcore_barrier()
            fill_rows(1.0)

            def deg_scatter(b):
                pltpu.sync_copy(rows0, agg_sh.at[didx[b]], add=True)

            idx_start(0, 0).start()

            def deg_pair(j, carry):
                c0 = j * 2
                idx_start(c0 + 1, 1).start()
                idx_start(c0, 0).wait()
                deg_scatter(0)

                @pl.when(c0 + 2 < n_c)
                def _():
                    idx_start(c0 + 2, 0).start()

                idx_start(c0 + 1, 1).wait()
                deg_scatter(1)
                return carry

            lax.fori_loop(0, _BASE_CHUNKS // 2, deg_pair, 0)

            @pl.when(has_extra)
            def _():
                idx_start(_BASE_CHUNKS, 0).wait()
                deg_scatter(0)

            plsc.subcore_barrier()
            copy_slab_out(deg_out)

    return pl.kernel(body, mesh=mesh, out_type=out_type, scratch_types=scratch)


_sc_agg_deg = _make_sc_agg(with_deg=True)
_sc_agg = _make_sc_agg(with_deg=False)

_BLK = 1000
_GRID = _N // _BLK


def _lin2_body(x_ref, wl_ref, wr_ref, b_ref, y_ref, r_ref):
    xb = x_ref[...]
    y_ref[...] = jnp.dot(xb, wl_ref[...], preferred_element_type=jnp.float32)
    r_ref[...] = jnp.dot(xb, wr_ref[...],
                         preferred_element_type=jnp.float32) + b_ref[...]


def _mid_body(a0_ref, a1_ref, d0_ref, d1_ref, r1_ref, wl_ref, wr_ref, b_ref,
              y2_ref, r2_ref):
    deg = jnp.maximum(d0_ref[0, :, 0:1] + d1_ref[0, :, 0:1], 1.0)
    h = jnp.maximum((a0_ref[0] + a1_ref[0]) / deg + r1_ref[...], 0.0)
    y2_ref[...] = jnp.dot(h, wl_ref[...], preferred_element_type=jnp.float32)
    r2_ref[...] = jnp.dot(h, wr_ref[...],
                          preferred_element_type=jnp.float32) + b_ref[...]


def _out_body(a0_ref, a1_ref, d0_ref, d1_ref, r2_ref, z_ref):
    deg = jnp.maximum(d0_ref[0, :, 0:1] + d1_ref[0, :, 0:1], 1.0)
    z_ref[...] = (a0_ref[0] + a1_ref[0]) / deg + r2_ref[...]


def _row_spec(w=_D):
    return pl.BlockSpec((_BLK, w), lambda i: (i, 0))


def _half_spec(half):
    return pl.BlockSpec((1, _BLK, _D), lambda i, h=half: (h, i, 0))


def _full_spec(shape):
    return pl.BlockSpec(shape, lambda i: (0, 0))


_f32 = jnp.float32
_lin2 = pl.pallas_call(
    _lin2_body,
    grid=(_GRID,),
    in_specs=[_row_spec(), _full_spec((_D, _D)), _full_spec((_D, _D)),
              _full_spec((1, _D))],
    out_specs=[_row_spec(), _row_spec()],
    out_shape=[jax.ShapeDtypeStruct((_N, _D), _f32)] * 2,
)

_mid = pl.pallas_call(
    _mid_body,
    grid=(_GRID,),
    in_specs=[_half_spec(0), _half_spec(1), _half_spec(0), _half_spec(1),
              _row_spec(), _full_spec((_D, _D)), _full_spec((_D, _D)),
              _full_spec((1, _D))],
    out_specs=[_row_spec(), _row_spec()],
    out_shape=[jax.ShapeDtypeStruct((_N, _D), _f32)] * 2,
)

_out = pl.pallas_call(
    _out_body,
    grid=(_GRID,),
    in_specs=[_half_spec(0), _half_spec(1), _half_spec(0), _half_spec(1),
              _row_spec()],
    out_specs=_row_spec(),
    out_shape=jax.ShapeDtypeStruct((_N, _D), _f32),
)


def kernel(x, edge_index, Wl1, bl1, Wr1, Wl2, bl2, Wr2):
    src = edge_index[0]
    dst = edge_index[1]
    y1, r1 = _lin2(x, Wl1.T, Wr1.T, bl1.reshape(1, _D))
    agg1, deg2 = _sc_agg_deg(y1, src, dst)
    y2, r2 = _mid(agg1, agg1, deg2, deg2, r1,
                  Wl2.T, Wr2.T, bl2.reshape(1, _D))
    agg2 = _sc_agg(y2, src, dst)
    if isinstance(agg2, (list, tuple)):
        agg2 = agg2[0]
    return _out(agg2, agg2, deg2, deg2, r2)


# speedup vs baseline: 13.1640x; 1.1896x over previous
import functools

import jax
import jax.numpy as jnp
from jax import lax
from jax.experimental import pallas as pl
from jax.experimental.pallas import tpu as pltpu
from jax.experimental.pallas import tpu_sc as plsc

_N = 10000
_E = 320000
_D = 128

_NTILES = 32
_NSUB = 16
_N_PAD = 10240
_ROWS_PER_TILE = _N_PAD // _NSUB
_CHUNK = 128
_TOT_CHUNKS = _E // _CHUNK
_BASE_CHUNKS = _TOT_CHUNKS // _NTILES
_XTRA_TILES = _TOT_CHUNKS - _BASE_CHUNKS * _NTILES


def _make_sc_agg(with_deg: bool):
    mesh = plsc.VectorSubcoreMesh(core_axis_name="c", subcore_axis_name="s")
    out_type = [jax.ShapeDtypeStruct((2, _N_PAD, _D), jnp.float32)]
    scratch = [
        pltpu.VMEM(((_BASE_CHUNKS + 1) * _CHUNK,), jnp.int32),
        pltpu.VMEM((_CHUNK,), jnp.int32),
        pltpu.VMEM((_CHUNK,), jnp.int32),
        pltpu.VMEM((_CHUNK, _D), jnp.float32),
        pltpu.VMEM((_CHUNK, _D), jnp.float32),
        pltpu.VMEM_SHARED((_N_PAD, _D), jnp.float32),
        pltpu.SemaphoreType.DMA,
        pltpu.SemaphoreType.DMA,
        pltpu.SemaphoreType.DMA,
        pltpu.SemaphoreType.DMA,
    ]
    if with_deg:
        out_type.append(jax.ShapeDtypeStruct((2, _N_PAD, _D), jnp.float32))
        scratch.append(pltpu.VMEM((_CHUNK,), jnp.float32))
        scratch.append(pltpu.VMEM((_ROWS_PER_TILE,), jnp.float32))
        scratch.append(pltpu.VMEM_SHARED((_N_PAD,), jnp.float32))

    def body(y, src, dst, *rest):
        if with_deg:
            agg_out, deg_out = rest[:2]
            rest = rest[2:]
            (sbuf, didx0, didx1, rows0, rows1, agg_sh, sg0, sg1, si0, si1,
             ones_v, dvec, deg_all) = rest
        else:
            agg_out = rest[0]
            sbuf, didx0, didx1, rows0, rows1, agg_sh, sg0, sg1, si0, si1 = \
                rest[1:]
        didx = (didx0, didx1)
        rows = (rows0, rows1)
        sg = (sg0, sg1)
        si = (si0, si1)
        cid = lax.axis_index("c")
        sid = lax.axis_index("s")
        tid = cid * _NSUB + sid
        chunk_base = tid * _BASE_CHUNKS + jnp.minimum(tid, _XTRA_TILES)
        n_c = _BASE_CHUNKS + (tid < _XTRA_TILES).astype(jnp.int32)
        has_extra = tid < _XTRA_TILES
        ebase = chunk_base * _CHUNK
        slab = sid * _ROWS_PER_TILE

        def fill_rows(val):
            def fill_row(r, carry):
                for c in range(_D // 16):
                    rows0[r, pl.ds(c * 16, 16)] = jnp.full((16,), val,
                                                           jnp.float32)
                return carry

            lax.fori_loop(0, _CHUNK, fill_row, 0)

        def zero_slab():
            for j in range(_ROWS_PER_TILE // _CHUNK):
                pltpu.sync_copy(rows0,
                                agg_sh.at[pl.ds(slab + j * _CHUNK, _CHUNK), :])

        def copy_slab_out(dst_hbm):
            pltpu.sync_copy(agg_sh.at[pl.ds(slab, _ROWS_PER_TILE), :],
                            dst_hbm.at[cid, pl.ds(slab, _ROWS_PER_TILE), :])

        def idx_start(c, b):
            base = pl.multiple_of(ebase + c * _CHUNK, 8)
            return pltpu.make_async_copy(dst.at[pl.ds(base, _CHUNK)],
                                         didx[b], si[b])

        def gather_start(c, b):
            sl = pl.multiple_of(c * _CHUNK, 8)
            return pltpu.make_async_copy(
                y.at[sbuf.at[pl.ds(sl, _CHUNK)]], rows[b], sg[b])

        def scatter(b):
            pltpu.sync_copy(rows[b], agg_sh.at[didx[b]], add=True)
            if with_deg:
                pltpu.sync_copy(ones_v, deg_all.at[didx[b]], add=True)

        fill_rows(0.0)
        zero_slab()
        if with_deg:
            def zero_dvec(i, carry):
                dvec[pl.ds(i * 16, 16)] = jnp.zeros((16,), jnp.float32)
                return carry

            lax.fori_loop(0, _ROWS_PER_TILE // 16, zero_dvec, 0)
            pltpu.sync_copy(dvec, deg_all.at[pl.ds(slab, _ROWS_PER_TILE)])

            def one_row(i, carry):
                ones_v[pl.ds(i * 16, 16)] = jnp.full((16,), 1.0, jnp.float32)
                return carry

            lax.fori_loop(0, _CHUNK // 16, one_row, 0)
        nbulk = _BASE_CHUNKS * _CHUNK
        pltpu.sync_copy(src.at[pl.ds(pl.multiple_of(ebase, 8), nbulk)],
                        sbuf.at[pl.ds(0, nbulk)])

        @pl.when(has_extra)
        def _():
            pltpu.sync_copy(
                src.at[pl.ds(pl.multiple_of(ebase + nbulk, 8), _CHUNK)],
                sbuf.at[pl.ds(nbulk, _CHUNK)])

        plsc.subcore_barrier()

        idx_start(0, 0).start()
        gather_start(0, 0).start()

        def pair(j, carry):
            c0 = j * 2
            idx_start(c0 + 1, 1).start()
            gather_start(c0 + 1, 1).start()
            gather_start(c0, 0).wait()
            idx_start(c0, 0).wait()
            scatter(0)

            @pl.when(c0 + 2 < n_c)
            def _():
                idx_start(c0 + 2, 0).start()
                gather_start(c0 + 2, 0).start()

            gather_start(c0 + 1, 1).wait()
            idx_start(c0 + 1, 1).wait()
            scatter(1)
            return carry

        lax.fori_loop(0, _BASE_CHUNKS // 2, pair, 0)

        @pl.when(has_extra)
        def _():
            gather_start(_BASE_CHUNKS, 0).wait()
            idx_start(_BASE_CHUNKS, 0).wait()
            scatter(0)

        plsc.subcore_barrier()
        copy_slab_out(agg_out)

        if with_deg:
            pltpu.sync_copy(deg_all.at[pl.ds(slab, _ROWS_PER_TILE)], dvec)

            def deg_batch(bi, carry):
                for g in range(8):
                    s = dvec[pl.ds((bi * 8 + g) * 16, 16)]
                    for l in range(16):
                        val = s[l]
                        for c in range(_D // 16):
                            rows0[g * 16 + l, pl.ds(c * 16, 16)] = jnp.full(
                                (16,), val, jnp.float32)
                pltpu.sync_copy(
                    rows0,
                    deg_out.at[cid, pl.ds(slab + bi * _CHUNK, _CHUNK), :])
                return carry

            lax.fori_loop(0, _ROWS_PER_TILE // _CHUNK, deg_batch, 0)

    return pl.kernel(body, mesh=mesh, out_type=out_type, scratch_types=scratch)


_sc_agg_deg = _make_sc_agg(with_deg=True)
_sc_agg = _make_sc_agg(with_deg=False)

_BLK = 1000
_GRID = _N // _BLK


def _lin2_body(x_ref, wl_ref, wr_ref, b_ref, y_ref, r_ref):
    xb = x_ref[...]
    y_ref[...] = jnp.dot(xb, wl_ref[...], preferred_element_type=jnp.float32)
    r_ref[...] = jnp.dot(xb, wr_ref[...],
                         preferred_element_type=jnp.float32) + b_ref[...]


def _mid_body(a0_ref, a1_ref, d0_ref, d1_ref, r1_ref, wl_ref, wr_ref, b_ref,
              y2_ref, r2_ref):
    deg = jnp.maximum(d0_ref[0, :, 0:1] + d1_ref[0, :, 0:1], 1.0)
    h = jnp.maximum((a0_ref[0] + a1_ref[0]) / deg + r1_ref[...], 0.0)
    y2_ref[...] = jnp.dot(h, wl_ref[...], preferred_element_type=jnp.float32)
    r2_ref[...] = jnp.dot(h, wr_ref[...],
                          preferred_element_type=jnp.float32) + b_ref[...]


def _out_body(a0_ref, a1_ref, d0_ref, d1_ref, r2_ref, z_ref):
    deg = jnp.maximum(d0_ref[0, :, 0:1] + d1_ref[0, :, 0:1], 1.0)
    z_ref[...] = (a0_ref[0] + a1_ref[0]) / deg + r2_ref[...]


def _row_spec(w=_D):
    return pl.BlockSpec((_BLK, w), lambda i: (i, 0))


def _half_spec(half):
    return pl.BlockSpec((1, _BLK, _D), lambda i, h=half: (h, i, 0))


def _full_spec(shape):
    return pl.BlockSpec(shape, lambda i: (0, 0))


_f32 = jnp.float32
_lin2 = pl.pallas_call(
    _lin2_body,
    grid=(_GRID,),
    in_specs=[_row_spec(), _full_spec((_D, _D)), _full_spec((_D, _D)),
              _full_spec((1, _D))],
    out_specs=[_row_spec(), _row_spec()],
    out_shape=[jax.ShapeDtypeStruct((_N, _D), _f32)] * 2,
)

_mid = pl.pallas_call(
    _mid_body,
    grid=(_GRID,),
    in_specs=[_half_spec(0), _half_spec(1), _half_spec(0), _half_spec(1),
              _row_spec(), _full_spec((_D, _D)), _full_spec((_D, _D)),
              _full_spec((1, _D))],
    out_specs=[_row_spec(), _row_spec()],
    out_shape=[jax.ShapeDtypeStruct((_N, _D), _f32)] * 2,
)

_out = pl.pallas_call(
    _out_body,
    grid=(_GRID,),
    in_specs=[_half_spec(0), _half_spec(1), _half_spec(0), _half_spec(1),
              _row_spec()],
    out_specs=_row_spec(),
    out_shape=jax.ShapeDtypeStruct((_N, _D), _f32),
)


def kernel(x, edge_index, Wl1, bl1, Wr1, Wl2, bl2, Wr2):
    src = edge_index[0]
    dst = edge_index[1]
    y1, r1 = _lin2(x, Wl1.T, Wr1.T, bl1.reshape(1, _D))
    agg1, deg2 = _sc_agg_deg(y1, src, dst)
    y2, r2 = _mid(agg1, agg1, deg2, deg2, r1,
                  Wl2.T, Wr2.T, bl2.reshape(1, _D))
    agg2 = _sc_agg(y2, src, dst)
    if isinstance(agg2, (list, tuple)):
        agg2 = agg2[0]
    return _out(agg2, agg2, deg2, deg2, r2)


# speedup vs baseline: 14.0462x; 1.0670x over previous
import functools

import jax
import jax.numpy as jnp
from jax import lax
from jax.experimental import pallas as pl
from jax.experimental.pallas import tpu as pltpu
from jax.experimental.pallas import tpu_sc as plsc

_N = 10000
_E = 320000
_D = 128

_NTILES = 32
_NSUB = 16
_N_PAD = 10240
_ROWS_PER_TILE = _N_PAD // _NSUB
_CHUNK = 128
_TOT_CHUNKS = _E // _CHUNK
_BASE_CHUNKS = _TOT_CHUNKS // _NTILES
_XTRA_TILES = _TOT_CHUNKS - _BASE_CHUNKS * _NTILES


def _make_sc_agg(with_deg: bool):
    mesh = plsc.VectorSubcoreMesh(core_axis_name="c", subcore_axis_name="s")
    out_type = [jax.ShapeDtypeStruct((2, _N_PAD, _D), jnp.float32)]
    scratch = [
        pltpu.VMEM(((_BASE_CHUNKS + 1) * _CHUNK,), jnp.int32),
        pltpu.VMEM((_CHUNK,), jnp.int32),
        pltpu.VMEM((_CHUNK,), jnp.int32),
        pltpu.VMEM((_CHUNK, _D), jnp.float32),
        pltpu.VMEM((_CHUNK, _D), jnp.float32),
        pltpu.VMEM_SHARED((_N_PAD, _D), jnp.float32),
        pltpu.SemaphoreType.DMA,
        pltpu.SemaphoreType.DMA,
        pltpu.SemaphoreType.DMA,
        pltpu.SemaphoreType.DMA,
    ]
    if with_deg:
        out_type.append(jax.ShapeDtypeStruct((2, _N_PAD, _D), jnp.float32))
        scratch.append(pltpu.VMEM((_CHUNK,), jnp.float32))
        scratch.append(pltpu.VMEM((_ROWS_PER_TILE,), jnp.float32))
        scratch.append(pltpu.VMEM_SHARED((_N_PAD,), jnp.float32))

    def body(y, eidx, *rest):
        if with_deg:
            agg_out, deg_out = rest[:2]
            rest = rest[2:]
            (sbuf, didx0, didx1, rows0, rows1, agg_sh, sg0, sg1, si0, si1,
             ones_v, dvec, deg_all) = rest
        else:
            agg_out = rest[0]
            sbuf, didx0, didx1, rows0, rows1, agg_sh, sg0, sg1, si0, si1 = \
                rest[1:]
        didx = (didx0, didx1)
        rows = (rows0, rows1)
        sg = (sg0, sg1)
        si = (si0, si1)
        cid = lax.axis_index("c")
        sid = lax.axis_index("s")
        tid = cid * _NSUB + sid
        chunk_base = tid * _BASE_CHUNKS + jnp.minimum(tid, _XTRA_TILES)
        n_c = _BASE_CHUNKS + (tid < _XTRA_TILES).astype(jnp.int32)
        has_extra = tid < _XTRA_TILES
        ebase = chunk_base * _CHUNK
        slab = sid * _ROWS_PER_TILE

        def fill_rows(val):
            def fill_row(r, carry):
                for c in range(_D // 16):
                    rows0[r, pl.ds(c * 16, 16)] = jnp.full((16,), val,
                                                           jnp.float32)
                return carry

            lax.fori_loop(0, _CHUNK, fill_row, 0)

        def zero_slab():
            for j in range(_ROWS_PER_TILE // _CHUNK):
                pltpu.sync_copy(rows0,
                                agg_sh.at[pl.ds(slab + j * _CHUNK, _CHUNK), :])

        def copy_slab_out(dst_hbm):
            pltpu.sync_copy(agg_sh.at[pl.ds(slab, _ROWS_PER_TILE), :],
                            dst_hbm.at[cid, pl.ds(slab, _ROWS_PER_TILE), :])

        def idx_start(c, b):
            base = pl.multiple_of(ebase + c * _CHUNK, 8)
            return pltpu.make_async_copy(eidx.at[1, pl.ds(base, _CHUNK)],
                                         didx[b], si[b])

        def gather_start(c, b):
            sl = pl.multiple_of(c * _CHUNK, 8)
            return pltpu.make_async_copy(
                y.at[sbuf.at[pl.ds(sl, _CHUNK)]], rows[b], sg[b])

        def scatter(b):
            pltpu.sync_copy(rows[b], agg_sh.at[didx[b]], add=True)
            if with_deg:
                pltpu.sync_copy(ones_v, deg_all.at[didx[b]], add=True)

        fill_rows(0.0)
        zero_slab()
        if with_deg:
            def zero_dvec(i, carry):
                dvec[pl.ds(i * 16, 16)] = jnp.zeros((16,), jnp.float32)
                return carry

            lax.fori_loop(0, _ROWS_PER_TILE // 16, zero_dvec, 0)
            pltpu.sync_copy(dvec, deg_all.at[pl.ds(slab, _ROWS_PER_TILE)])

            def one_row(i, carry):
                ones_v[pl.ds(i * 16, 16)] = jnp.full((16,), 1.0, jnp.float32)
                return carry

            lax.fori_loop(0, _CHUNK // 16, one_row, 0)
        nbulk = _BASE_CHUNKS * _CHUNK
        pltpu.sync_copy(eidx.at[0, pl.ds(pl.multiple_of(ebase, 8), nbulk)],
                        sbuf.at[pl.ds(0, nbulk)])

        @pl.when(has_extra)
        def _():
            pltpu.sync_copy(
                eidx.at[0, pl.ds(pl.multiple_of(ebase + nbulk, 8), _CHUNK)],
                sbuf.at[pl.ds(nbulk, _CHUNK)])

        plsc.subcore_barrier()

        idx_start(0, 0).start()
        gather_start(0, 0).start()

        def pair(j, carry):
            c0 = j * 2
            idx_start(c0 + 1, 1).start()
            gather_start(c0 + 1, 1).start()
            gather_start(c0, 0).wait()
            idx_start(c0, 0).wait()
            scatter(0)

            @pl.when(c0 + 2 < n_c)
            def _():
                idx_start(c0 + 2, 0).start()
                gather_start(c0 + 2, 0).start()

            gather_start(c0 + 1, 1).wait()
            idx_start(c0 + 1, 1).wait()
            scatter(1)
            return carry

        lax.fori_loop(0, _BASE_CHUNKS // 2, pair, 0)

        @pl.when(has_extra)
        def _():
            gather_start(_BASE_CHUNKS, 0).wait()
            idx_start(_BASE_CHUNKS, 0).wait()
            scatter(0)

        plsc.subcore_barrier()
        copy_slab_out(agg_out)

        if with_deg:
            pltpu.sync_copy(deg_all.at[pl.ds(slab, _ROWS_PER_TILE)], dvec)

            def deg_batch(bi, carry):
                for g in range(8):
                    s = dvec[pl.ds((bi * 8 + g) * 16, 16)]
                    for l in range(16):
                        val = s[l]
                        for c in range(_D // 16):
                            rows0[g * 16 + l, pl.ds(c * 16, 16)] = jnp.full(
                                (16,), val, jnp.float32)
                pltpu.sync_copy(
                    rows0,
                    deg_out.at[cid, pl.ds(slab + bi * _CHUNK, _CHUNK), :])
                return carry

            lax.fori_loop(0, _ROWS_PER_TILE // _CHUNK, deg_batch, 0)

    return pl.kernel(body, mesh=mesh, out_type=out_type, scratch_types=scratch)


_sc_agg_deg = _make_sc_agg(with_deg=True)
_sc_agg = _make_sc_agg(with_deg=False)

_BLK = 1000
_GRID = _N // _BLK


def _matT(a, w):
    return lax.dot_general(a, w, (((1,), (1,)), ((), ())),
                           preferred_element_type=jnp.float32)


def _lin2_body(x_ref, wl_ref, wr_ref, b_ref, y_ref, r_ref):
    xb = x_ref[...]
    y_ref[...] = _matT(xb, wl_ref[...])
    r_ref[...] = _matT(xb, wr_ref[...]) + b_ref[...]


def _mid_body(a0_ref, a1_ref, d0_ref, d1_ref, r1_ref, wl_ref, wr_ref, b_ref,
              y2_ref, r2_ref):
    deg = jnp.maximum(d0_ref[0, :, 0:1] + d1_ref[0, :, 0:1], 1.0)
    h = jnp.maximum((a0_ref[0] + a1_ref[0]) / deg + r1_ref[...], 0.0)
    y2_ref[...] = _matT(h, wl_ref[...])
    r2_ref[...] = _matT(h, wr_ref[...]) + b_ref[...]


def _out_body(a0_ref, a1_ref, d0_ref, d1_ref, r2_ref, z_ref):
    deg = jnp.maximum(d0_ref[0, :, 0:1] + d1_ref[0, :, 0:1], 1.0)
    z_ref[...] = (a0_ref[0] + a1_ref[0]) / deg + r2_ref[...]


def _row_spec(w=_D):
    return pl.BlockSpec((_BLK, w), lambda i: (i, 0))


def _half_spec(half):
    return pl.BlockSpec((1, _BLK, _D), lambda i, h=half: (h, i, 0))


def _full_spec(shape):
    return pl.BlockSpec(shape, lambda i: (0, 0))


_f32 = jnp.float32
_lin2 = pl.pallas_call(
    _lin2_body,
    grid=(_GRID,),
    in_specs=[_row_spec(), _full_spec((_D, _D)), _full_spec((_D, _D)),
              _full_spec((1, _D))],
    out_specs=[_row_spec(), _row_spec()],
    out_shape=[jax.ShapeDtypeStruct((_N, _D), _f32)] * 2,
)

_mid = pl.pallas_call(
    _mid_body,
    grid=(_GRID,),
    in_specs=[_half_spec(0), _half_spec(1), _half_spec(0), _half_spec(1),
              _row_spec(), _full_spec((_D, _D)), _full_spec((_D, _D)),
              _full_spec((1, _D))],
    out_specs=[_row_spec(), _row_spec()],
    out_shape=[jax.ShapeDtypeStruct((_N, _D), _f32)] * 2,
)

_out = pl.pallas_call(
    _out_body,
    grid=(_GRID,),
    in_specs=[_half_spec(0), _half_spec(1), _half_spec(0), _half_spec(1),
              _row_spec()],
    out_specs=_row_spec(),
    out_shape=jax.ShapeDtypeStruct((_N, _D), _f32),
)


def kernel(x, edge_index, Wl1, bl1, Wr1, Wl2, bl2, Wr2):
    y1, r1 = _lin2(x, Wl1, Wr1, bl1.reshape(1, _D))
    agg1, deg2 = _sc_agg_deg(y1, edge_index)
    y2, r2 = _mid(agg1, agg1, deg2, deg2, r1,
                  Wl2, Wr2, bl2.reshape(1, _D))
    agg2 = _sc_agg(y2, edge_index)
    if isinstance(agg2, (list, tuple)):
        agg2 = agg2[0]
    return _out(agg2, agg2, deg2, deg2, r2)
